# baseline (speedup 1.0000x reference)
import jax
import jax.numpy as jnp
from jax.experimental import pallas as pl
from jax.experimental.pallas import tpu as pltpu

_TEMPERATURE = 64.0
_N_SPLIT = 4


def _protonet_body(s_ref, q_ref, o_ref):
    inv_t = 1.0 / _TEMPERATURE
    proto = jnp.sum(s_ref[...], axis=0) * (1.0 / s_ref.shape[0])
    q = q_ref[...]
    qn = jnp.sum(q * q, axis=1, keepdims=True) * inv_t
    pn = (jnp.sum(proto * proto, axis=1) * inv_t)[None, :]
    cross = jax.lax.dot_general(
        q, proto * (2.0 * inv_t), (((1,), (1,)), ((), ())),
        preferred_element_type=jnp.float32,
    )
    o_ref[...] = cross - qn - pn


def kernel(support, query):
    n_batch, n_shot, n_way, emb_dim = support.shape
    n_query = n_batch * query.shape[1] * n_way
    blk = n_query // _N_SPLIT
    s = support.reshape(n_shot, n_way, emb_dim)
    q = query.reshape(n_query, emb_dim)
    return pl.pallas_call(
        _protonet_body,
        grid=(_N_SPLIT,),
        in_specs=[
            pl.BlockSpec((n_shot, n_way, emb_dim), lambda i: (0, 0, 0)),
            pl.BlockSpec((blk, emb_dim), lambda i: (i, 0)),
        ],
        out_specs=pl.BlockSpec((blk, n_way), lambda i: (i, 0)),
        out_shape=jax.ShapeDtypeStruct((n_query, n_way), jnp.float32),
        compiler_params=pltpu.CompilerParams(
            dimension_semantics=("parallel",),
        ),
    )(s, q)


# speedup vs baseline: 1.2424x; 1.2424x over previous
import jax
import jax.numpy as jnp
from jax.experimental import pallas as pl
from jax.experimental.pallas import tpu as pltpu

_TEMPERATURE = 64.0
_N_SPLIT = 2


def _protonet_body(s_ref, q_ref, o_ref):
    inv_t = 1.0 / _TEMPERATURE
    proto = jnp.sum(s_ref[...], axis=0) * (1.0 / s_ref.shape[0])
    q = q_ref[...]
    acc = q[:, :128] * q[:, :128]
    for k in range(128, q.shape[1], 128):
        sl = q[:, k:k + 128]
        acc = acc + sl * sl
    qn = jnp.sum(acc, axis=1, keepdims=True) * inv_t
    pn = (jnp.sum(proto * proto, axis=1) * inv_t)[None, :]
    cross = jax.lax.dot_general(
        q, proto * (2.0 * inv_t), (((1,), (1,)), ((), ())),
        preferred_element_type=jnp.float32,
    )
    o_ref[...] = cross - qn - pn


def kernel(support, query):
    n_batch, n_shot, n_way, emb_dim = support.shape
    n_query = n_batch * query.shape[1] * n_way
    blk = n_query // _N_SPLIT
    s = support.reshape(n_shot, n_way, emb_dim)
    q = query.reshape(n_query, emb_dim)
    return pl.pallas_call(
        _protonet_body,
        grid=(_N_SPLIT,),
        in_specs=[
            pl.BlockSpec((n_shot, n_way, emb_dim), lambda i: (0, 0, 0)),
            pl.BlockSpec((blk, emb_dim), lambda i: (i, 0)),
        ],
        out_specs=pl.BlockSpec((blk, n_way), lambda i: (i, 0)),
        out_shape=jax.ShapeDtypeStruct((n_query, n_way), jnp.float32),
        compiler_params=pltpu.CompilerParams(
            dimension_semantics=("parallel",),
        ),
    )(s, q)
